# Initial kernel scaffold; baseline (speedup 1.0000x reference)
import functools

import jax
import jax.numpy as jnp
from jax import lax
from jax.experimental import pallas as pl
from jax.experimental.pallas import tpu as pltpu
from jax.experimental.pallas import tpu_sc as plsc

_NE = 8192
_ED = 32
_NTOK = 16384
_TOK_BLK = 256

_NC = 2
_NS = 16
_NW = _NC * _NS
_B_PER_W = _NTOK // _NW
_CH = 128
_NCH = _B_PER_W // _CH


def _dist_argmin_body(z_ref, t1_ref, embt_ref, t2_ref, idx_ref):
    mm = jnp.dot(z_ref[...], embt_ref[...], preferred_element_type=jnp.float32)
    d = (t1_ref[...] + t2_ref[...]) - 2.0 * mm
    m = jnp.min(d, axis=1, keepdims=True)
    iota = lax.broadcasted_iota(jnp.int32, d.shape, 1)
    idx_ref[...] = jnp.min(jnp.where(d == m, iota, jnp.int32(2**30)), axis=1)


def _dist_argmin(z_flat, t1, embt, t2):
    return pl.pallas_call(
        _dist_argmin_body,
        grid=(_NTOK // _TOK_BLK,),
        in_specs=[
            pl.BlockSpec((_TOK_BLK, _ED), lambda i: (i, 0)),
            pl.BlockSpec((_TOK_BLK, 1), lambda i: (i, 0)),
            pl.BlockSpec((_ED, _NE), lambda i: (0, 0)),
            pl.BlockSpec((1, _NE), lambda i: (0, 0)),
        ],
        out_specs=pl.BlockSpec((_TOK_BLK,), lambda i: (i,)),
        out_shape=jax.ShapeDtypeStruct((_NTOK,), jnp.int32),
    )(z_flat, t1, embt, t2)


def _sc_gather(emb_weight, idx2d):
    mesh = plsc.VectorSubcoreMesh(core_axis_name="c", subcore_axis_name="s")

    @functools.partial(
        pl.kernel,
        mesh=mesh,
        out_type=jax.ShapeDtypeStruct((_NTOK, _ED), jnp.float32),
        scratch_types=[
            pltpu.VMEM((_NCH, _CH), jnp.int32),
            pltpu.VMEM((_B_PER_W, _ED), jnp.float32),
            pltpu.SemaphoreType.DMA,
        ],
    )
    def k(table_hbm, idx_hbm, out_hbm, idx_v, rows_v, sem):
        wid = lax.axis_index("s") * _NC + lax.axis_index("c")
        pltpu.sync_copy(idx_hbm.at[pl.ds(wid * _NCH, _NCH)], idx_v)
        for c in range(_NCH):
            pltpu.async_copy(
                table_hbm.at[idx_v.at[c]], rows_v.at[pl.ds(c * _CH, _CH)], sem
            ).wait()
        pltpu.sync_copy(rows_v, out_hbm.at[pl.ds(wid * _B_PER_W, _B_PER_W)])

    return k(emb_weight, idx2d)


def kernel(z, emb_weight):
    z_flat = z.reshape(-1, _ED)
    t1 = jnp.sum(z_flat**2, axis=1, keepdims=True)
    t2 = jnp.sum(emb_weight**2, axis=1)[None, :]
    embt = emb_weight.T
    idx = _dist_argmin(z_flat, t1, embt, t2)
    zq_rows = _sc_gather(emb_weight, idx.reshape(_NTOK // _CH, _CH))
    z_q = zq_rows.reshape(z.shape)
    z_q = z + lax.stop_gradient(z_q - z)
    return (z_q, idx)


# speedup vs baseline: 1.2782x; 1.2782x over previous
import functools

import jax
import jax.numpy as jnp
from jax import lax
from jax.experimental import pallas as pl
from jax.experimental.pallas import tpu as pltpu
from jax.experimental.pallas import tpu_sc as plsc

_NE = 8192
_ED = 32
_NTOK = 16384
_TOK_BLK = 256

_NC = 2
_NS = 16
_NW = _NC * _NS
_B_PER_W = _NTOK // _NW
_CH = 128
_NCH = _B_PER_W // _CH


def _dist_argmin_body(z_ref, t1_ref, embt_ref, t2_ref, idx_ref):
    mm = jnp.dot(z_ref[...], embt_ref[...], preferred_element_type=jnp.float32)
    d = (t1_ref[...] + t2_ref[...]) - 2.0 * mm
    m = jnp.min(d, axis=1, keepdims=True)
    iota = lax.broadcasted_iota(jnp.int32, d.shape, 1)
    idx_ref[...] = jnp.min(jnp.where(d == m, iota, jnp.int32(2**30)), axis=1)


def _dist_argmin(z_flat, t1, embt, t2):
    return pl.pallas_call(
        _dist_argmin_body,
        grid=(_NTOK // _TOK_BLK,),
        in_specs=[
            pl.BlockSpec((_TOK_BLK, _ED), lambda i: (i, 0)),
            pl.BlockSpec((_TOK_BLK, 1), lambda i: (i, 0)),
            pl.BlockSpec((_ED, _NE), lambda i: (0, 0)),
            pl.BlockSpec((1, _NE), lambda i: (0, 0)),
        ],
        out_specs=pl.BlockSpec((_TOK_BLK,), lambda i: (i,)),
        out_shape=jax.ShapeDtypeStruct((_NTOK,), jnp.int32),
    )(z_flat, t1, embt, t2)


def _sc_gather(emb_weight, idx2d):
    mesh = plsc.VectorSubcoreMesh(core_axis_name="c", subcore_axis_name="s")

    @functools.partial(
        pl.kernel,
        mesh=mesh,
        out_type=jax.ShapeDtypeStruct((_NTOK, _ED), jnp.float32),
        scratch_types=[
            pltpu.VMEM((_NCH, _CH), jnp.int32),
            pltpu.VMEM((_B_PER_W, _ED), jnp.float32),
            pltpu.SemaphoreType.DMA,
        ],
        compiler_params=pltpu.CompilerParams(use_tc_tiling_on_sc=False),
    )
    def k(table_hbm, idx_hbm, out_hbm, idx_v, rows_v, sem):
        wid = lax.axis_index("s") * _NC + lax.axis_index("c")
        pltpu.sync_copy(idx_hbm.at[pl.ds(wid * _NCH, _NCH)], idx_v)
        for c in range(_NCH):
            pltpu.async_copy(
                table_hbm.at[idx_v.at[c]], rows_v.at[pl.ds(c * _CH, _CH)], sem
            ).wait()
        pltpu.sync_copy(rows_v, out_hbm.at[pl.ds(wid * _B_PER_W, _B_PER_W)])

    return k(emb_weight, idx2d)


def kernel(z, emb_weight):
    z_flat = z.reshape(-1, _ED)
    t1 = jnp.sum(z_flat**2, axis=1, keepdims=True)
    t2 = jnp.sum(emb_weight**2, axis=1)[None, :]
    embt = emb_weight.T
    idx = _dist_argmin(z_flat, t1, embt, t2)
    zq_rows = _sc_gather(emb_weight, idx.reshape(_NTOK // _CH, _CH))
    z_q = zq_rows.reshape(z.shape)
    z_q = z + lax.stop_gradient(z_q - z)
    return (z_q, idx)


# speedup vs baseline: 1.3205x; 1.0331x over previous
import functools

import jax
import jax.numpy as jnp
from jax import lax
from jax.experimental import pallas as pl
from jax.experimental.pallas import tpu as pltpu
from jax.experimental.pallas import tpu_sc as plsc

_NE = 8192
_ED = 32
_NTOK = 16384
_TOK_BLK = 512

_NC = 2
_NS = 16
_NW = _NC * _NS
_B_PER_W = _NTOK // _NW
_CH = 128
_NCH = _B_PER_W // _CH


def _dist_argmin_body(z_ref, t1_ref, embt_ref, t2_ref, idx_ref):
    mm = jnp.dot(z_ref[...], embt_ref[...], preferred_element_type=jnp.float32)
    d = (t1_ref[...] + t2_ref[...]) - 2.0 * mm
    m = jnp.min(d, axis=1, keepdims=True)
    iota = lax.broadcasted_iota(jnp.int32, d.shape, 1)
    idx_ref[...] = jnp.min(jnp.where(d == m, iota, jnp.int32(2**30)), axis=1)


def _dist_argmin(z_flat, t1, embt, t2):
    return pl.pallas_call(
        _dist_argmin_body,
        grid=(_NTOK // _TOK_BLK,),
        in_specs=[
            pl.BlockSpec((_TOK_BLK, _ED), lambda i: (i, 0)),
            pl.BlockSpec((_TOK_BLK, 1), lambda i: (i, 0)),
            pl.BlockSpec((_ED, _NE), lambda i: (0, 0)),
            pl.BlockSpec((1, _NE), lambda i: (0, 0)),
        ],
        out_specs=pl.BlockSpec((_TOK_BLK,), lambda i: (i,)),
        out_shape=jax.ShapeDtypeStruct((_NTOK,), jnp.int32),
    )(z_flat, t1, embt, t2)


def _sc_gather(emb_weight, idx2d):
    mesh = plsc.VectorSubcoreMesh(core_axis_name="c", subcore_axis_name="s")

    @functools.partial(
        pl.kernel,
        mesh=mesh,
        out_type=jax.ShapeDtypeStruct((_NTOK, _ED), jnp.float32),
        scratch_types=[
            pltpu.VMEM((_NCH, _CH), jnp.int32),
            pltpu.VMEM((_B_PER_W, _ED), jnp.float32),
            pltpu.SemaphoreType.DMA,
        ],
        compiler_params=pltpu.CompilerParams(use_tc_tiling_on_sc=False),
    )
    def k(table_hbm, idx_hbm, out_hbm, idx_v, rows_v, sem):
        wid = lax.axis_index("s") * _NC + lax.axis_index("c")
        pltpu.sync_copy(idx_hbm.at[pl.ds(wid * _NCH, _NCH)], idx_v)
        for c in range(_NCH):
            pltpu.async_copy(
                table_hbm.at[idx_v.at[c]], rows_v.at[pl.ds(c * _CH, _CH)], sem
            ).wait()
        pltpu.sync_copy(rows_v, out_hbm.at[pl.ds(wid * _B_PER_W, _B_PER_W)])

    return k(emb_weight, idx2d)


def kernel(z, emb_weight):
    z_flat = z.reshape(-1, _ED)
    t1 = jnp.sum(z_flat**2, axis=1, keepdims=True)
    t2 = jnp.sum(emb_weight**2, axis=1)[None, :]
    embt = emb_weight.T
    idx = _dist_argmin(z_flat, t1, embt, t2)
    zq_rows = _sc_gather(emb_weight, idx.reshape(_NTOK // _CH, _CH))
    z_q = zq_rows.reshape(z.shape)
    z_q = z + lax.stop_gradient(z_q - z)
    return (z_q, idx)
